# Initial kernel scaffold; baseline (speedup 1.0000x reference)
import jax
import jax.numpy as jnp
from jax.experimental import pallas as pl


def kernel(x, edge_index, batch, atom_emb, W1, b1, g1, be1, W2, b2, bn_g, bn_b, Wh, bh, Wo, bo):
    raise NotImplementedError("write your pallas kernel here")


# speedup vs baseline: 2.6131x; 2.6131x over previous
import functools

import jax
import jax.numpy as jnp
from jax import lax
from jax.experimental import pallas as pl
from jax.experimental.pallas import tpu as pltpu
from jax.experimental.pallas import tpu_sc as plsc

N = 10000
E = 320000
NF = 9
VOCAB = 119
HID = 128
G = 256
LAYERS = 5

NPAD = 10240
NW = 32
NCORES = 2
FPW = HID // NW
ECH = 3200
NECH = E // ECH
EUNROLL = 4

CB = 2048
NB = NPAD // CB


def _sc_segsum_body(hT_hbm, edges_hbm, out_hbm, h_v, agg_v, idx_v):
    wid = lax.axis_index("s") * NCORES + lax.axis_index("c")
    base0 = wid * (FPW * NPAD)
    pltpu.sync_copy(hT_hbm.at[pl.ds(base0, FPW * NPAD)], h_v)

    def zero_body(i, carry):
        agg_v[pl.ds(i * 16, 16)] = jnp.zeros((16,), jnp.float32)
        return carry

    lax.fori_loop(0, FPW * NPAD // 16, zero_body, 0)

    def chunk_body(c, carry):
        pltpu.sync_copy(edges_hbm.at[:, pl.ds(c * ECH, ECH)], idx_v)

        def edge_body(i, icarry):
            base = i * (16 * EUNROLL)
            for u in range(EUNROLL):
                src = idx_v[0, pl.ds(base + u * 16, 16)]
                dst = idx_v[1, pl.ds(base + u * 16, 16)]
                for r in range(FPW):
                    vals = plsc.load_gather(h_v, [src + (r * NPAD)])
                    plsc.addupdate_scatter(agg_v, [dst + (r * NPAD)], vals)
            return icarry

        lax.fori_loop(0, ECH // (16 * EUNROLL), edge_body, 0)
        return carry

    lax.fori_loop(0, NECH, chunk_body, 0)
    pltpu.sync_copy(agg_v, out_hbm.at[pl.ds(base0, FPW * NPAD)])


@functools.lru_cache(maxsize=None)
def _get_sc_segsum():
    return pl.kernel(
        _sc_segsum_body,
        out_type=jax.ShapeDtypeStruct((HID * NPAD,), jnp.float32),
        mesh=plsc.VectorSubcoreMesh(core_axis_name="c", subcore_axis_name="s"),
        compiler_params=pltpu.CompilerParams(needs_layout_passes=False),
        scratch_types=[
            pltpu.VMEM((FPW * NPAD,), jnp.float32),
            pltpu.VMEM((FPW * NPAD,), jnp.float32),
            pltpu.VMEM((2, ECH), jnp.int32),
        ],
    )


def _atom_body(xT_ref, embT_ref, out_ref):
    iota = lax.broadcasted_iota(jnp.int32, (HID, CB), 0)
    acc = jnp.zeros((HID, CB), jnp.float32)
    for f in range(NF):
        onehot = (iota == xT_ref[f, :][None, :]).astype(jnp.float32)
        acc = acc + jnp.dot(embT_ref[f], onehot,
                            preferred_element_type=jnp.float32)
    out_ref[...] = acc


def _atom_encode(xT, embT):
    return pl.pallas_call(
        _atom_body,
        grid=(NB,),
        in_specs=[
            pl.BlockSpec((16, CB), lambda i: (0, i)),
            pl.BlockSpec((NF, HID, HID), lambda i: (0, 0, 0)),
        ],
        out_specs=pl.BlockSpec((HID, CB), lambda i: (0, i)),
        out_shape=jax.ShapeDtypeStruct((HID, NPAD), jnp.float32),
    )(xT, embT)


def _mlp_body(h_ref, a_ref, w1_ref, b1_ref, w2_ref, b2_ref, out_ref, *, final):
    z = h_ref[...] + a_ref[...]
    z = jnp.dot(w1_ref[...], z, preferred_element_type=jnp.float32) + b1_ref[...]
    z = jnp.maximum(z, 0.0)
    z = jnp.dot(w2_ref[...], z, preferred_element_type=jnp.float32) + b2_ref[...]
    if not final:
        z = jnp.maximum(z, 0.0)
    out_ref[...] = z


def _mlp(h, agg, w1, b1, w2, b2, final):
    return pl.pallas_call(
        functools.partial(_mlp_body, final=final),
        grid=(NB,),
        in_specs=[
            pl.BlockSpec((HID, CB), lambda i: (0, i)),
            pl.BlockSpec((HID, CB), lambda i: (0, i)),
            pl.BlockSpec((HID, HID), lambda i: (0, 0)),
            pl.BlockSpec((HID, 1), lambda i: (0, 0)),
            pl.BlockSpec((HID, HID), lambda i: (0, 0)),
            pl.BlockSpec((HID, 1), lambda i: (0, 0)),
        ],
        out_specs=pl.BlockSpec((HID, CB), lambda i: (0, i)),
        out_shape=jax.ShapeDtypeStruct((HID, NPAD), jnp.float32),
    )(h, agg, w1, b1, w2, b2)


def _pool_body(h_ref, b_ref, out_ref):
    @pl.when(pl.program_id(0) == 0)
    def _():
        out_ref[...] = jnp.zeros_like(out_ref)

    iota = lax.broadcasted_iota(jnp.int32, (G, CB), 0)
    onehotT = (iota == b_ref[0, :, :]).astype(jnp.float32)
    out_ref[...] += lax.dot_general(
        onehotT, h_ref[...], (((1,), (1,)), ((), ())),
        preferred_element_type=jnp.float32)


def _pool(h, batch3):
    return pl.pallas_call(
        _pool_body,
        grid=(NB,),
        in_specs=[
            pl.BlockSpec((HID, CB), lambda i: (0, i)),
            pl.BlockSpec((1, 1, CB), lambda i: (i, 0, 0)),
        ],
        out_specs=pl.BlockSpec((G, HID), lambda i: (0, 0)),
        out_shape=jax.ShapeDtypeStruct((G, HID), jnp.float32),
    )(h, batch3)


def _head_body(p_ref, whT_ref, bh_ref, woT_ref, bo_ref, out_ref):
    a = jnp.dot(p_ref[...], whT_ref[...], preferred_element_type=jnp.float32)
    a = jnp.maximum(a + bh_ref[...], 0.0)
    out_ref[...] = jnp.dot(a, woT_ref[...],
                           preferred_element_type=jnp.float32) + bo_ref[...]


def _head(pooled, whT, bh2, woT, bo2):
    return pl.pallas_call(
        _head_body,
        in_specs=[
            pl.BlockSpec((G, HID), lambda: (0, 0)),
            pl.BlockSpec((HID, HID), lambda: (0, 0)),
            pl.BlockSpec((1, HID), lambda: (0, 0)),
            pl.BlockSpec((HID, HID), lambda: (0, 0)),
            pl.BlockSpec((1, HID), lambda: (0, 0)),
        ],
        out_specs=pl.BlockSpec((G, HID), lambda: (0, 0)),
        out_shape=jax.ShapeDtypeStruct((G, HID), jnp.float32),
    )(pooled, whT, bh2, woT, bo2)


def kernel(x, edge_index, batch, atom_emb, W1, b1, g1, be1, W2, b2,
           bn_g, bn_b, Wh, bh, Wo, bo):
    xT = jnp.pad(x, ((0, NPAD - N), (0, 0))).T
    xT = jnp.pad(xT, ((0, 16 - NF), (0, 0)))
    embT = jnp.pad(jnp.transpose(atom_emb, (0, 2, 1)),
                   ((0, 0), (0, 0), (0, HID - VOCAB)))
    batch3 = jnp.pad(batch, (0, NPAD - N),
                     constant_values=G + 1).reshape(NB, 1, CB)

    W1f = g1[:, :, None] * W1
    b1f = (b1 * g1 + be1)[:, :, None]
    scale2 = jnp.concatenate([bn_g, jnp.ones((1, HID), jnp.float32)], 0)
    shift2 = jnp.concatenate([bn_b, jnp.zeros((1, HID), jnp.float32)], 0)
    W2f = scale2[:, :, None] * W2
    b2f = (b2 * scale2 + shift2)[:, :, None]

    h = _atom_encode(xT, embT)
    for l in range(LAYERS):
        agg = _get_sc_segsum()(h.reshape(-1), edge_index).reshape(HID, NPAD)
        h = _mlp(h, agg, W1f[l], b1f[l], W2f[l], b2f[l], final=(l == LAYERS - 1))
    pooled = _pool(h, batch3)
    return _head(pooled, Wh.T, bh[None, :], Wo.T, bo[None, :])


# speedup vs baseline: 5.0392x; 1.9284x over previous
import functools

import jax
import jax.numpy as jnp
from jax import lax
from jax.experimental import pallas as pl
from jax.experimental.pallas import tpu as pltpu
from jax.experimental.pallas import tpu_sc as plsc

N = 10000
E = 320000
NF = 9
VOCAB = 119
HID = 128
G = 256
LAYERS = 5

NPAD = 10240
NW = 32
NCORES = 2
FPW = HID // NW
ECH = 3200
NECH = E // ECH
EUNROLL = 4

CB = 2048
NB = NPAD // CB


def _sc_segsum_body(hT_hbm, edges_hbm, out_hbm, h_v, agg_v, idx_v):
    wid = lax.axis_index("s") * NCORES + lax.axis_index("c")
    base0 = wid * (FPW * NPAD)
    pltpu.sync_copy(hT_hbm.at[pl.ds(base0, FPW * NPAD)], h_v)

    def zero_body(i, carry):
        agg_v[pl.ds(i * 16, 16)] = jnp.zeros((16,), jnp.float32)
        return carry

    lax.fori_loop(0, FPW * NPAD // 16, zero_body, 0)

    def chunk_body(c, carry):
        pltpu.sync_copy(edges_hbm.at[:, pl.ds(c * ECH, ECH)], idx_v)

        def edge_body(i, icarry):
            base = i * (16 * EUNROLL)
            srcs = [idx_v[0, pl.ds(base + u * 16, 16)] for u in range(EUNROLL)]
            dsts = [idx_v[1, pl.ds(base + u * 16, 16)] for u in range(EUNROLL)]
            vals = [plsc.load_gather(h_v, [srcs[u] + (r * NPAD)])
                    for u in range(EUNROLL) for r in range(FPW)]
            for u in range(EUNROLL):
                for r in range(FPW):
                    plsc.addupdate_scatter(agg_v, [dsts[u] + (r * NPAD)],
                                           vals[u * FPW + r])
            return icarry

        lax.fori_loop(0, ECH // (16 * EUNROLL), edge_body, 0)
        return carry

    lax.fori_loop(0, NECH, chunk_body, 0)
    pltpu.sync_copy(agg_v, out_hbm.at[pl.ds(base0, FPW * NPAD)])


@functools.lru_cache(maxsize=None)
def _get_sc_segsum():
    return pl.kernel(
        _sc_segsum_body,
        out_type=jax.ShapeDtypeStruct((HID * NPAD,), jnp.float32),
        mesh=plsc.VectorSubcoreMesh(core_axis_name="c", subcore_axis_name="s"),
        compiler_params=pltpu.CompilerParams(needs_layout_passes=False),
        scratch_types=[
            pltpu.VMEM((FPW * NPAD,), jnp.float32),
            pltpu.VMEM((FPW * NPAD,), jnp.float32),
            pltpu.VMEM((2, ECH), jnp.int32),
        ],
    )


def _atom_body(xT_ref, embT_ref, out_ref):
    iota = lax.broadcasted_iota(jnp.int32, (HID, CB), 0)
    acc = jnp.zeros((HID, CB), jnp.float32)
    for f in range(NF):
        onehot = (iota == xT_ref[f, :][None, :]).astype(jnp.float32)
        acc = acc + jnp.dot(embT_ref[f], onehot,
                            preferred_element_type=jnp.float32)
    out_ref[...] = acc


def _atom_encode(xT, embT):
    return pl.pallas_call(
        _atom_body,
        grid=(NB,),
        in_specs=[
            pl.BlockSpec((16, CB), lambda i: (0, i)),
            pl.BlockSpec((NF, HID, HID), lambda i: (0, 0, 0)),
        ],
        out_specs=pl.BlockSpec((HID, CB), lambda i: (0, i)),
        out_shape=jax.ShapeDtypeStruct((HID, NPAD), jnp.float32),
    )(xT, embT)


def _mlp_body(h_ref, a_ref, w1_ref, b1_ref, w2_ref, b2_ref, out_ref, *, final):
    z = h_ref[...] + a_ref[...]
    z = jnp.dot(w1_ref[...], z, preferred_element_type=jnp.float32) + b1_ref[...]
    z = jnp.maximum(z, 0.0)
    z = jnp.dot(w2_ref[...], z, preferred_element_type=jnp.float32) + b2_ref[...]
    if not final:
        z = jnp.maximum(z, 0.0)
    out_ref[...] = z


def _mlp(h, agg, w1, b1, w2, b2, final):
    return pl.pallas_call(
        functools.partial(_mlp_body, final=final),
        grid=(NB,),
        in_specs=[
            pl.BlockSpec((HID, CB), lambda i: (0, i)),
            pl.BlockSpec((HID, CB), lambda i: (0, i)),
            pl.BlockSpec((HID, HID), lambda i: (0, 0)),
            pl.BlockSpec((HID, 1), lambda i: (0, 0)),
            pl.BlockSpec((HID, HID), lambda i: (0, 0)),
            pl.BlockSpec((HID, 1), lambda i: (0, 0)),
        ],
        out_specs=pl.BlockSpec((HID, CB), lambda i: (0, i)),
        out_shape=jax.ShapeDtypeStruct((HID, NPAD), jnp.float32),
    )(h, agg, w1, b1, w2, b2)


def _pool_body(h_ref, b_ref, out_ref):
    @pl.when(pl.program_id(0) == 0)
    def _():
        out_ref[...] = jnp.zeros_like(out_ref)

    iota = lax.broadcasted_iota(jnp.int32, (G, CB), 0)
    onehotT = (iota == b_ref[0, :, :]).astype(jnp.float32)
    out_ref[...] += lax.dot_general(
        onehotT, h_ref[...], (((1,), (1,)), ((), ())),
        preferred_element_type=jnp.float32)


def _pool(h, batch3):
    return pl.pallas_call(
        _pool_body,
        grid=(NB,),
        in_specs=[
            pl.BlockSpec((HID, CB), lambda i: (0, i)),
            pl.BlockSpec((1, 1, CB), lambda i: (i, 0, 0)),
        ],
        out_specs=pl.BlockSpec((G, HID), lambda i: (0, 0)),
        out_shape=jax.ShapeDtypeStruct((G, HID), jnp.float32),
    )(h, batch3)


def _head_body(p_ref, whT_ref, bh_ref, woT_ref, bo_ref, out_ref):
    a = jnp.dot(p_ref[...], whT_ref[...], preferred_element_type=jnp.float32)
    a = jnp.maximum(a + bh_ref[...], 0.0)
    out_ref[...] = jnp.dot(a, woT_ref[...],
                           preferred_element_type=jnp.float32) + bo_ref[...]


def _head(pooled, whT, bh2, woT, bo2):
    return pl.pallas_call(
        _head_body,
        in_specs=[
            pl.BlockSpec((G, HID), lambda: (0, 0)),
            pl.BlockSpec((HID, HID), lambda: (0, 0)),
            pl.BlockSpec((1, HID), lambda: (0, 0)),
            pl.BlockSpec((HID, HID), lambda: (0, 0)),
            pl.BlockSpec((1, HID), lambda: (0, 0)),
        ],
        out_specs=pl.BlockSpec((G, HID), lambda: (0, 0)),
        out_shape=jax.ShapeDtypeStruct((G, HID), jnp.float32),
    )(pooled, whT, bh2, woT, bo2)


def kernel(x, edge_index, batch, atom_emb, W1, b1, g1, be1, W2, b2,
           bn_g, bn_b, Wh, bh, Wo, bo):
    xT = jnp.pad(x, ((0, NPAD - N), (0, 0))).T
    xT = jnp.pad(xT, ((0, 16 - NF), (0, 0)))
    embT = jnp.pad(jnp.transpose(atom_emb, (0, 2, 1)),
                   ((0, 0), (0, 0), (0, HID - VOCAB)))
    batch3 = jnp.pad(batch, (0, NPAD - N),
                     constant_values=G + 1).reshape(NB, 1, CB)

    W1f = g1[:, :, None] * W1
    b1f = (b1 * g1 + be1)[:, :, None]
    scale2 = jnp.concatenate([bn_g, jnp.ones((1, HID), jnp.float32)], 0)
    shift2 = jnp.concatenate([bn_b, jnp.zeros((1, HID), jnp.float32)], 0)
    W2f = scale2[:, :, None] * W2
    b2f = (b2 * scale2 + shift2)[:, :, None]

    h = _atom_encode(xT, embT)
    for l in range(LAYERS):
        agg = _get_sc_segsum()(h.reshape(-1), edge_index).reshape(HID, NPAD)
        h = _mlp(h, agg, W1f[l], b1f[l], W2f[l], b2f[l], final=(l == LAYERS - 1))
    pooled = _pool(h, batch3)
    return _head(pooled, Wh.T, bh[None, :], Wo.T, bo[None, :])


# speedup vs baseline: 6.8861x; 1.3665x over previous
import functools

import jax
import jax.numpy as jnp
from jax import lax
from jax.experimental import pallas as pl
from jax.experimental.pallas import tpu as pltpu
from jax.experimental.pallas import tpu_sc as plsc

N = 10000
E = 320000
NF = 9
VOCAB = 119
HID = 128
G = 256
LAYERS = 5

NPAD = 10240
NW = 32
NCORES = 2
FPW = HID // NW
ECH = 3200
NECH = E // ECH
EUNROLL = 4

CB = 2048
NB = NPAD // CB


def _sc_segsum_body(hT_hbm, edges_hbm, out_hbm, h_v, agg_v, idx0_v, idx1_v,
                    sem0, sem1):
    wid = lax.axis_index("s") * NCORES + lax.axis_index("c")
    base0 = wid * (FPW * NPAD)
    pltpu.sync_copy(hT_hbm.at[pl.ds(base0, FPW * NPAD)], h_v)

    def zero_body(i, carry):
        agg_v[pl.ds(i * 16, 16)] = jnp.zeros((16,), jnp.float32)
        return carry

    lax.fori_loop(0, FPW * NPAD // 16, zero_body, 0)

    def start(c, buf, sem):
        pltpu.async_copy(edges_hbm.at[:, pl.ds(c * ECH, ECH)], buf, sem)

    def wait(buf, sem):
        pltpu.make_async_copy(edges_hbm.at[:, pl.ds(0, ECH)], buf, sem).wait()

    def process(idx_v):
        def edge_body(i, icarry):
            base = i * (16 * EUNROLL)
            srcs = [idx_v[0, pl.ds(base + u * 16, 16)] for u in range(EUNROLL)]
            dsts = [idx_v[1, pl.ds(base + u * 16, 16)] for u in range(EUNROLL)]
            vals = [plsc.load_gather(h_v, [srcs[u] + (r * NPAD)])
                    for u in range(EUNROLL) for r in range(FPW)]
            for u in range(EUNROLL):
                for r in range(FPW):
                    plsc.addupdate_scatter(agg_v, [dsts[u] + (r * NPAD)],
                                           vals[u * FPW + r])
            return icarry

        lax.fori_loop(0, ECH // (16 * EUNROLL), edge_body, 0)

    start(0, idx0_v, sem0)
    start(1, idx1_v, sem1)

    def chunk_body(c2, carry):
        wait(idx0_v, sem0)
        process(idx0_v)

        @pl.when(c2 < NECH // 2 - 1)
        def _():
            start(2 * c2 + 2, idx0_v, sem0)

        wait(idx1_v, sem1)
        process(idx1_v)

        @pl.when(c2 < NECH // 2 - 1)
        def _():
            start(2 * c2 + 3, idx1_v, sem1)

        return carry

    lax.fori_loop(0, NECH // 2, chunk_body, 0)
    pltpu.sync_copy(agg_v, out_hbm.at[pl.ds(base0, FPW * NPAD)])


@functools.lru_cache(maxsize=None)
def _get_sc_segsum():
    return pl.kernel(
        _sc_segsum_body,
        out_type=jax.ShapeDtypeStruct((HID * NPAD,), jnp.float32),
        mesh=plsc.VectorSubcoreMesh(core_axis_name="c", subcore_axis_name="s"),
        compiler_params=pltpu.CompilerParams(needs_layout_passes=False),
        scratch_types=[
            pltpu.VMEM((FPW * NPAD,), jnp.float32),
            pltpu.VMEM((FPW * NPAD,), jnp.float32),
            pltpu.VMEM((2, ECH), jnp.int32),
            pltpu.VMEM((2, ECH), jnp.int32),
            pltpu.SemaphoreType.DMA,
            pltpu.SemaphoreType.DMA,
        ],
    )


def _atom_body(xT_ref, embT_ref, out_ref):
    iota = lax.broadcasted_iota(jnp.int32, (HID, CB), 0)
    acc = jnp.zeros((HID, CB), jnp.float32)
    for f in range(NF):
        onehot = (iota == xT_ref[f, :][None, :]).astype(jnp.float32)
        acc = acc + jnp.dot(embT_ref[f], onehot,
                            preferred_element_type=jnp.float32)
    out_ref[...] = acc


def _atom_encode(xT, embT):
    return pl.pallas_call(
        _atom_body,
        grid=(NB,),
        in_specs=[
            pl.BlockSpec((16, CB), lambda i: (0, i)),
            pl.BlockSpec((NF, HID, HID), lambda i: (0, 0, 0)),
        ],
        out_specs=pl.BlockSpec((HID, CB), lambda i: (0, i)),
        out_shape=jax.ShapeDtypeStruct((HID, NPAD), jnp.float32),
    )(xT, embT)


def _mlp_body(h_ref, a_ref, w1_ref, b1_ref, w2_ref, b2_ref, out_ref, *, final):
    z = h_ref[...] + a_ref[...]
    z = jnp.dot(w1_ref[...], z, preferred_element_type=jnp.float32) + b1_ref[...]
    z = jnp.maximum(z, 0.0)
    z = jnp.dot(w2_ref[...], z, preferred_element_type=jnp.float32) + b2_ref[...]
    if not final:
        z = jnp.maximum(z, 0.0)
    out_ref[...] = z


def _mlp(h, agg, w1, b1, w2, b2, final):
    return pl.pallas_call(
        functools.partial(_mlp_body, final=final),
        grid=(NB,),
        in_specs=[
            pl.BlockSpec((HID, CB), lambda i: (0, i)),
            pl.BlockSpec((HID, CB), lambda i: (0, i)),
            pl.BlockSpec((HID, HID), lambda i: (0, 0)),
            pl.BlockSpec((HID, 1), lambda i: (0, 0)),
            pl.BlockSpec((HID, HID), lambda i: (0, 0)),
            pl.BlockSpec((HID, 1), lambda i: (0, 0)),
        ],
        out_specs=pl.BlockSpec((HID, CB), lambda i: (0, i)),
        out_shape=jax.ShapeDtypeStruct((HID, NPAD), jnp.float32),
    )(h, agg, w1, b1, w2, b2)


def _pool_body(h_ref, b_ref, out_ref):
    @pl.when(pl.program_id(0) == 0)
    def _():
        out_ref[...] = jnp.zeros_like(out_ref)

    iota = lax.broadcasted_iota(jnp.int32, (G, CB), 0)
    onehotT = (iota == b_ref[0, :, :]).astype(jnp.float32)
    out_ref[...] += lax.dot_general(
        onehotT, h_ref[...], (((1,), (1,)), ((), ())),
        preferred_element_type=jnp.float32)


def _pool(h, batch3):
    return pl.pallas_call(
        _pool_body,
        grid=(NB,),
        in_specs=[
            pl.BlockSpec((HID, CB), lambda i: (0, i)),
            pl.BlockSpec((1, 1, CB), lambda i: (i, 0, 0)),
        ],
        out_specs=pl.BlockSpec((G, HID), lambda i: (0, 0)),
        out_shape=jax.ShapeDtypeStruct((G, HID), jnp.float32),
    )(h, batch3)


def _head_body(p_ref, whT_ref, bh_ref, woT_ref, bo_ref, out_ref):
    a = jnp.dot(p_ref[...], whT_ref[...], preferred_element_type=jnp.float32)
    a = jnp.maximum(a + bh_ref[...], 0.0)
    out_ref[...] = jnp.dot(a, woT_ref[...],
                           preferred_element_type=jnp.float32) + bo_ref[...]


def _head(pooled, whT, bh2, woT, bo2):
    return pl.pallas_call(
        _head_body,
        in_specs=[
            pl.BlockSpec((G, HID), lambda: (0, 0)),
            pl.BlockSpec((HID, HID), lambda: (0, 0)),
            pl.BlockSpec((1, HID), lambda: (0, 0)),
            pl.BlockSpec((HID, HID), lambda: (0, 0)),
            pl.BlockSpec((1, HID), lambda: (0, 0)),
        ],
        out_specs=pl.BlockSpec((G, HID), lambda: (0, 0)),
        out_shape=jax.ShapeDtypeStruct((G, HID), jnp.float32),
    )(pooled, whT, bh2, woT, bo2)


def kernel(x, edge_index, batch, atom_emb, W1, b1, g1, be1, W2, b2,
           bn_g, bn_b, Wh, bh, Wo, bo):
    xT = jnp.pad(x, ((0, NPAD - N), (0, 0))).T
    xT = jnp.pad(xT, ((0, 16 - NF), (0, 0)))
    embT = jnp.pad(jnp.transpose(atom_emb, (0, 2, 1)),
                   ((0, 0), (0, 0), (0, HID - VOCAB)))
    batch3 = jnp.pad(batch, (0, NPAD - N),
                     constant_values=G + 1).reshape(NB, 1, CB)

    W1f = g1[:, :, None] * W1
    b1f = (b1 * g1 + be1)[:, :, None]
    scale2 = jnp.concatenate([bn_g, jnp.ones((1, HID), jnp.float32)], 0)
    shift2 = jnp.concatenate([bn_b, jnp.zeros((1, HID), jnp.float32)], 0)
    W2f = scale2[:, :, None] * W2
    b2f = (b2 * scale2 + shift2)[:, :, None]

    h = _atom_encode(xT, embT)
    for l in range(LAYERS):
        agg = _get_sc_segsum()(h.reshape(-1), edge_index).reshape(HID, NPAD)
        h = _mlp(h, agg, W1f[l], b1f[l], W2f[l], b2f[l], final=(l == LAYERS - 1))
    pooled = _pool(h, batch3)
    return _head(pooled, Wh.T, bh[None, :], Wo.T, bo[None, :])


# speedup vs baseline: 7.9434x; 1.1535x over previous
import functools

import jax
import jax.numpy as jnp
from jax import lax
from jax.experimental import pallas as pl
from jax.experimental.pallas import tpu as pltpu
from jax.experimental.pallas import tpu_sc as plsc

N = 10000
E = 320000
NF = 9
VOCAB = 119
HID = 128
G = 256
LAYERS = 5

NPAD = 10240
NW = 32
NCORES = 2
FPW = 8
PKW = FPW // 2
EHALF = E // 2
ECH = 640
NECH = EHALF // ECH
EUNROLL = 4

CB = 2048
NB = NPAD // CB


def _sc_segsum_body(hpk_hbm, edges_hbm, out_hbm, h_v, agg_v, idx0_v, idx1_v,
                    sem0, sem1):
    wid = lax.axis_index("s") * NCORES + lax.axis_index("c")
    fg = wid // 2
    half = wid % 2
    pltpu.sync_copy(hpk_hbm.at[pl.ds(fg * (PKW * NPAD), PKW * NPAD)], h_v)

    def zero_body(i, carry):
        for u in range(4):
            agg_v[pl.ds(i * 64 + u * 16, 16)] = jnp.zeros((16,), jnp.float32)
        return carry

    lax.fori_loop(0, FPW * NPAD // 64, zero_body, 0)

    ebase = half * EHALF

    def start(c, buf, sem):
        pltpu.async_copy(edges_hbm.at[:, pl.ds(ebase + c * ECH, ECH)], buf,
                         sem)

    def wait(buf, sem):
        pltpu.make_async_copy(edges_hbm.at[:, pl.ds(0, ECH)], buf, sem).wait()

    def process(idx_v):
        def edge_body(i, icarry):
            base = i * (16 * EUNROLL)
            srcs = [idx_v[0, pl.ds(base + u * 16, 16)] for u in range(EUNROLL)]
            dsts = [idx_v[1, pl.ds(base + u * 16, 16)] for u in range(EUNROLL)]
            words = [plsc.load_gather(h_v, [srcs[u] + (p * NPAD)])
                     for u in range(EUNROLL) for p in range(PKW)]
            for u in range(EUNROLL):
                for p in range(PKW):
                    wi = plsc.bitcast(words[u * PKW + p], jnp.int32)
                    lo = plsc.bitcast(wi << 16, jnp.float32)
                    hi = plsc.bitcast(wi & jnp.int32(-65536), jnp.float32)
                    plsc.addupdate_scatter(
                        agg_v, [dsts[u] + (2 * p * NPAD)], lo)
                    plsc.addupdate_scatter(
                        agg_v, [dsts[u] + ((2 * p + 1) * NPAD)], hi)
            return icarry

        lax.fori_loop(0, ECH // (16 * EUNROLL), edge_body, 0)

    start(0, idx0_v, sem0)
    start(1, idx1_v, sem1)

    def chunk_body(c2, carry):
        wait(idx0_v, sem0)
        process(idx0_v)

        @pl.when(c2 < NECH // 2 - 1)
        def _():
            start(2 * c2 + 2, idx0_v, sem0)

        wait(idx1_v, sem1)
        process(idx1_v)

        @pl.when(c2 < NECH // 2 - 1)
        def _():
            start(2 * c2 + 3, idx1_v, sem1)

        return carry

    lax.fori_loop(0, NECH // 2, chunk_body, 0)
    out_off = half * (HID * NPAD) + fg * (FPW * NPAD)
    pltpu.sync_copy(agg_v, out_hbm.at[pl.ds(out_off, FPW * NPAD)])


@functools.lru_cache(maxsize=None)
def _get_sc_segsum():
    return pl.kernel(
        _sc_segsum_body,
        out_type=jax.ShapeDtypeStruct((2 * HID * NPAD,), jnp.float32),
        mesh=plsc.VectorSubcoreMesh(core_axis_name="c", subcore_axis_name="s"),
        compiler_params=pltpu.CompilerParams(needs_layout_passes=False),
        scratch_types=[
            pltpu.VMEM((PKW * NPAD,), jnp.float32),
            pltpu.VMEM((FPW * NPAD,), jnp.float32),
            pltpu.VMEM((2, ECH), jnp.int32),
            pltpu.VMEM((2, ECH), jnp.int32),
            pltpu.SemaphoreType.DMA,
            pltpu.SemaphoreType.DMA,
        ],
    )


def _pack_pairs(z):
    zb = z.astype(jnp.bfloat16).reshape(HID // 2, 2, CB)
    u = lax.bitcast_convert_type(zb, jnp.uint16).astype(jnp.uint32)
    w = u[:, 0, :] | (u[:, 1, :] << jnp.uint32(16))
    return lax.bitcast_convert_type(w, jnp.float32)


def _atom_body(xT_ref, embT_ref, out_ref, pk_ref):
    iota = lax.broadcasted_iota(jnp.int32, (HID, CB), 0)
    acc = jnp.zeros((HID, CB), jnp.float32)
    for f in range(NF):
        onehot = (iota == xT_ref[f, :][None, :]).astype(jnp.float32)
        acc = acc + jnp.dot(embT_ref[f], onehot,
                            preferred_element_type=jnp.float32)
    out_ref[...] = acc
    pk_ref[...] = _pack_pairs(acc)


def _atom_encode(xT, embT):
    return pl.pallas_call(
        _atom_body,
        grid=(NB,),
        in_specs=[
            pl.BlockSpec((16, CB), lambda i: (0, i)),
            pl.BlockSpec((NF, HID, HID), lambda i: (0, 0, 0)),
        ],
        out_specs=[
            pl.BlockSpec((HID, CB), lambda i: (0, i)),
            pl.BlockSpec((HID // 2, CB), lambda i: (0, i)),
        ],
        out_shape=[
            jax.ShapeDtypeStruct((HID, NPAD), jnp.float32),
            jax.ShapeDtypeStruct((HID // 2, NPAD), jnp.float32),
        ],
    )(xT, embT)


def _mlp_body(h_ref, a_ref, w1_ref, b1_ref, w2_ref, b2_ref, out_ref, pk_ref,
              *, final):
    z = h_ref[...] + a_ref[0] + a_ref[1]
    z = jnp.dot(w1_ref[...], z, preferred_element_type=jnp.float32) + b1_ref[...]
    z = jnp.maximum(z, 0.0)
    z = jnp.dot(w2_ref[...], z, preferred_element_type=jnp.float32) + b2_ref[...]
    if not final:
        z = jnp.maximum(z, 0.0)
    out_ref[...] = z
    pk_ref[...] = _pack_pairs(z)


def _mlp(h, agg2, w1, b1, w2, b2, final):
    return pl.pallas_call(
        functools.partial(_mlp_body, final=final),
        grid=(NB,),
        in_specs=[
            pl.BlockSpec((HID, CB), lambda i: (0, i)),
            pl.BlockSpec((2, HID, CB), lambda i: (0, 0, i)),
            pl.BlockSpec((HID, HID), lambda i: (0, 0)),
            pl.BlockSpec((HID, 1), lambda i: (0, 0)),
            pl.BlockSpec((HID, HID), lambda i: (0, 0)),
            pl.BlockSpec((HID, 1), lambda i: (0, 0)),
        ],
        out_specs=[
            pl.BlockSpec((HID, CB), lambda i: (0, i)),
            pl.BlockSpec((HID // 2, CB), lambda i: (0, i)),
        ],
        out_shape=[
            jax.ShapeDtypeStruct((HID, NPAD), jnp.float32),
            jax.ShapeDtypeStruct((HID // 2, NPAD), jnp.float32),
        ],
    )(h, agg2, w1, b1, w2, b2)


def _pool_body(h_ref, b_ref, out_ref):
    @pl.when(pl.program_id(0) == 0)
    def _():
        out_ref[...] = jnp.zeros_like(out_ref)

    iota = lax.broadcasted_iota(jnp.int32, (G, CB), 0)
    onehotT = (iota == b_ref[0, :, :]).astype(jnp.float32)
    out_ref[...] += lax.dot_general(
        onehotT, h_ref[...], (((1,), (1,)), ((), ())),
        preferred_element_type=jnp.float32)


def _pool(h, batch3):
    return pl.pallas_call(
        _pool_body,
        grid=(NB,),
        in_specs=[
            pl.BlockSpec((HID, CB), lambda i: (0, i)),
            pl.BlockSpec((1, 1, CB), lambda i: (i, 0, 0)),
        ],
        out_specs=pl.BlockSpec((G, HID), lambda i: (0, 0)),
        out_shape=jax.ShapeDtypeStruct((G, HID), jnp.float32),
    )(h, batch3)


def _head_body(p_ref, whT_ref, bh_ref, woT_ref, bo_ref, out_ref):
    a = jnp.dot(p_ref[...], whT_ref[...], preferred_element_type=jnp.float32)
    a = jnp.maximum(a + bh_ref[...], 0.0)
    out_ref[...] = jnp.dot(a, woT_ref[...],
                           preferred_element_type=jnp.float32) + bo_ref[...]


def _head(pooled, whT, bh2, woT, bo2):
    return pl.pallas_call(
        _head_body,
        in_specs=[
            pl.BlockSpec((G, HID), lambda: (0, 0)),
            pl.BlockSpec((HID, HID), lambda: (0, 0)),
            pl.BlockSpec((1, HID), lambda: (0, 0)),
            pl.BlockSpec((HID, HID), lambda: (0, 0)),
            pl.BlockSpec((1, HID), lambda: (0, 0)),
        ],
        out_specs=pl.BlockSpec((G, HID), lambda: (0, 0)),
        out_shape=jax.ShapeDtypeStruct((G, HID), jnp.float32),
    )(pooled, whT, bh2, woT, bo2)


def kernel(x, edge_index, batch, atom_emb, W1, b1, g1, be1, W2, b2,
           bn_g, bn_b, Wh, bh, Wo, bo):
    xT = jnp.pad(x, ((0, NPAD - N), (0, 0))).T
    xT = jnp.pad(xT, ((0, 16 - NF), (0, 0)))
    embT = jnp.pad(jnp.transpose(atom_emb, (0, 2, 1)),
                   ((0, 0), (0, 0), (0, HID - VOCAB)))
    batch3 = jnp.pad(batch, (0, NPAD - N),
                     constant_values=G + 1).reshape(NB, 1, CB)

    W1f = g1[:, :, None] * W1
    b1f = (b1 * g1 + be1)[:, :, None]
    scale2 = jnp.concatenate([bn_g, jnp.ones((1, HID), jnp.float32)], 0)
    shift2 = jnp.concatenate([bn_b, jnp.zeros((1, HID), jnp.float32)], 0)
    W2f = scale2[:, :, None] * W2
    b2f = (b2 * scale2 + shift2)[:, :, None]

    h, hpk = _atom_encode(xT, embT)
    for l in range(LAYERS):
        agg2 = _get_sc_segsum()(hpk.reshape(-1),
                                edge_index).reshape(2, HID, NPAD)
        h, hpk = _mlp(h, agg2, W1f[l], b1f[l], W2f[l], b2f[l],
                      final=(l == LAYERS - 1))
    pooled = _pool(h, batch3)
    return _head(pooled, Wh.T, bh[None, :], Wo.T, bo[None, :])


# speedup vs baseline: 7.9568x; 1.0017x over previous
import functools

import jax
import jax.numpy as jnp
from jax import lax
from jax.experimental import pallas as pl
from jax.experimental.pallas import tpu as pltpu
from jax.experimental.pallas import tpu_sc as plsc

N = 10000
E = 320000
NF = 9
VOCAB = 119
HID = 128
G = 256
LAYERS = 5

NPAD = 10240
NW = 32
NCORES = 2
FPW = 8
PKW = FPW // 2
EHALF = E // 2
ECH = 640
NECH = EHALF // ECH
EUNROLL = 8

CB = 2048
NB = NPAD // CB


def _sc_segsum_body(hpk_hbm, edges_hbm, out_hbm, h_v, agg_v, idx0_v, idx1_v,
                    sem0, sem1, semh):
    wid = lax.axis_index("s") * NCORES + lax.axis_index("c")
    fg = wid // 2
    half = wid % 2
    ebase = half * EHALF

    def start(c, buf, sem):
        pltpu.async_copy(edges_hbm.at[:, pl.ds(ebase + c * ECH, ECH)], buf,
                         sem)

    def wait(buf, sem):
        pltpu.make_async_copy(edges_hbm.at[:, pl.ds(0, ECH)], buf, sem).wait()

    hsrc = hpk_hbm.at[pl.ds(fg * (PKW * NPAD), PKW * NPAD)]
    pltpu.async_copy(hsrc, h_v, semh)
    start(0, idx0_v, sem0)
    start(1, idx1_v, sem1)

    def zero_body(i, carry):
        for u in range(4):
            agg_v[pl.ds(i * 64 + u * 16, 16)] = jnp.zeros((16,), jnp.float32)
        return carry

    lax.fori_loop(0, FPW * NPAD // 64, zero_body, 0)
    pltpu.make_async_copy(hsrc, h_v, semh).wait()

    def process(idx_v):
        def edge_body(i, icarry):
            base = i * (16 * EUNROLL)
            srcs = [idx_v[0, pl.ds(base + u * 16, 16)] for u in range(EUNROLL)]
            dsts = [idx_v[1, pl.ds(base + u * 16, 16)] for u in range(EUNROLL)]
            words = [plsc.load_gather(h_v, [srcs[u] + (p * NPAD)])
                     for u in range(EUNROLL) for p in range(PKW)]
            for u in range(EUNROLL):
                for p in range(PKW):
                    wi = plsc.bitcast(words[u * PKW + p], jnp.int32)
                    lo = plsc.bitcast(wi << 16, jnp.float32)
                    hi = plsc.bitcast(wi & jnp.int32(-65536), jnp.float32)
                    plsc.addupdate_scatter(
                        agg_v, [dsts[u] + (2 * p * NPAD)], lo)
                    plsc.addupdate_scatter(
                        agg_v, [dsts[u] + ((2 * p + 1) * NPAD)], hi)
            return icarry

        lax.fori_loop(0, ECH // (16 * EUNROLL), edge_body, 0)

    def chunk_body(c2, carry):
        wait(idx0_v, sem0)
        process(idx0_v)

        @pl.when(c2 < NECH // 2 - 1)
        def _():
            start(2 * c2 + 2, idx0_v, sem0)

        wait(idx1_v, sem1)
        process(idx1_v)

        @pl.when(c2 < NECH // 2 - 1)
        def _():
            start(2 * c2 + 3, idx1_v, sem1)

        return carry

    lax.fori_loop(0, NECH // 2, chunk_body, 0)
    out_off = half * (HID * NPAD) + fg * (FPW * NPAD)
    pltpu.sync_copy(agg_v, out_hbm.at[pl.ds(out_off, FPW * NPAD)])


@functools.lru_cache(maxsize=None)
def _get_sc_segsum():
    return pl.kernel(
        _sc_segsum_body,
        out_type=jax.ShapeDtypeStruct((2 * HID * NPAD,), jnp.float32),
        mesh=plsc.VectorSubcoreMesh(core_axis_name="c", subcore_axis_name="s"),
        compiler_params=pltpu.CompilerParams(needs_layout_passes=False),
        scratch_types=[
            pltpu.VMEM((PKW * NPAD,), jnp.float32),
            pltpu.VMEM((FPW * NPAD,), jnp.float32),
            pltpu.VMEM((2, ECH), jnp.int32),
            pltpu.VMEM((2, ECH), jnp.int32),
            pltpu.SemaphoreType.DMA,
            pltpu.SemaphoreType.DMA,
            pltpu.SemaphoreType.DMA,
        ],
    )


def _pack_pairs(z):
    zb = z.astype(jnp.bfloat16).reshape(HID // 2, 2, CB)
    u = lax.bitcast_convert_type(zb, jnp.uint16).astype(jnp.uint32)
    w = u[:, 0, :] | (u[:, 1, :] << jnp.uint32(16))
    return lax.bitcast_convert_type(w, jnp.float32)


def _unpack_pairs(wpk):
    wi = lax.bitcast_convert_type(wpk, jnp.uint32)
    lo = lax.bitcast_convert_type(wi << jnp.uint32(16), jnp.float32)
    hi = lax.bitcast_convert_type(wi & jnp.uint32(0xFFFF0000), jnp.float32)
    return jnp.stack([lo, hi], axis=1).reshape(HID, CB)


def _atom_body(xT_ref, embT_ref, pk_ref):
    iota = lax.broadcasted_iota(jnp.int32, (HID, CB), 0)
    acc = jnp.zeros((HID, CB), jnp.float32)
    for f in range(NF):
        onehot = (iota == xT_ref[f, :][None, :]).astype(jnp.float32)
        acc = acc + jnp.dot(embT_ref[f], onehot,
                            preferred_element_type=jnp.float32)
    pk_ref[...] = _pack_pairs(acc)


def _atom_encode(xT, embT):
    return pl.pallas_call(
        _atom_body,
        grid=(NB,),
        in_specs=[
            pl.BlockSpec((16, CB), lambda i: (0, i)),
            pl.BlockSpec((NF, HID, HID), lambda i: (0, 0, 0)),
        ],
        out_specs=pl.BlockSpec((HID // 2, CB), lambda i: (0, i)),
        out_shape=jax.ShapeDtypeStruct((HID // 2, NPAD), jnp.float32),
    )(xT, embT)


def _mlp_body(hpk_ref, a_ref, w1_ref, b1_ref, w2_ref, b2_ref, out_ref,
              *, final):
    z = _unpack_pairs(hpk_ref[...]) + a_ref[0] + a_ref[1]
    z = jnp.dot(w1_ref[...], z, preferred_element_type=jnp.float32) + b1_ref[...]
    z = jnp.maximum(z, 0.0)
    z = jnp.dot(w2_ref[...], z, preferred_element_type=jnp.float32) + b2_ref[...]
    if final:
        out_ref[...] = z
    else:
        out_ref[...] = _pack_pairs(jnp.maximum(z, 0.0))


def _mlp(hpk, agg2, w1, b1, w2, b2, final):
    orows = HID if final else HID // 2
    return pl.pallas_call(
        functools.partial(_mlp_body, final=final),
        grid=(NB,),
        in_specs=[
            pl.BlockSpec((HID // 2, CB), lambda i: (0, i)),
            pl.BlockSpec((2, HID, CB), lambda i: (0, 0, i)),
            pl.BlockSpec((HID, HID), lambda i: (0, 0)),
            pl.BlockSpec((HID, 1), lambda i: (0, 0)),
            pl.BlockSpec((HID, HID), lambda i: (0, 0)),
            pl.BlockSpec((HID, 1), lambda i: (0, 0)),
        ],
        out_specs=pl.BlockSpec((orows, CB), lambda i: (0, i)),
        out_shape=jax.ShapeDtypeStruct((orows, NPAD), jnp.float32),
    )(hpk, agg2, w1, b1, w2, b2)


def _poolhead_body(h_ref, b_ref, whT_ref, bh_ref, woT_ref, bo_ref, out_ref,
                   acc_ref):
    @pl.when(pl.program_id(0) == 0)
    def _():
        acc_ref[...] = jnp.zeros_like(acc_ref)

    iota = lax.broadcasted_iota(jnp.int32, (G, CB), 0)
    onehotT = (iota == b_ref[0, :, :]).astype(jnp.float32)
    acc_ref[...] += lax.dot_general(
        onehotT, h_ref[...], (((1,), (1,)), ((), ())),
        preferred_element_type=jnp.float32)

    @pl.when(pl.program_id(0) == NB - 1)
    def _():
        a = jnp.dot(acc_ref[...], whT_ref[...],
                    preferred_element_type=jnp.float32)
        a = jnp.maximum(a + bh_ref[...], 0.0)
        out_ref[...] = jnp.dot(a, woT_ref[...],
                               preferred_element_type=jnp.float32) + bo_ref[...]


def _poolhead(h, batch3, whT, bh2, woT, bo2):
    return pl.pallas_call(
        _poolhead_body,
        grid=(NB,),
        in_specs=[
            pl.BlockSpec((HID, CB), lambda i: (0, i)),
            pl.BlockSpec((1, 1, CB), lambda i: (i, 0, 0)),
            pl.BlockSpec((HID, HID), lambda i: (0, 0)),
            pl.BlockSpec((1, HID), lambda i: (0, 0)),
            pl.BlockSpec((HID, HID), lambda i: (0, 0)),
            pl.BlockSpec((1, HID), lambda i: (0, 0)),
        ],
        out_specs=pl.BlockSpec((G, HID), lambda i: (0, 0)),
        out_shape=jax.ShapeDtypeStruct((G, HID), jnp.float32),
        scratch_shapes=[pltpu.VMEM((G, HID), jnp.float32)],
    )(h, batch3, whT, bh2, woT, bo2)


def kernel(x, edge_index, batch, atom_emb, W1, b1, g1, be1, W2, b2,
           bn_g, bn_b, Wh, bh, Wo, bo):
    xT = jnp.pad(x, ((0, NPAD - N), (0, 0))).T
    xT = jnp.pad(xT, ((0, 16 - NF), (0, 0)))
    embT = jnp.pad(jnp.transpose(atom_emb, (0, 2, 1)),
                   ((0, 0), (0, 0), (0, HID - VOCAB)))
    batch3 = jnp.pad(batch, (0, NPAD - N),
                     constant_values=G + 1).reshape(NB, 1, CB)

    W1f = g1[:, :, None] * W1
    b1f = (b1 * g1 + be1)[:, :, None]
    scale2 = jnp.concatenate([bn_g, jnp.ones((1, HID), jnp.float32)], 0)
    shift2 = jnp.concatenate([bn_b, jnp.zeros((1, HID), jnp.float32)], 0)
    W2f = scale2[:, :, None] * W2
    b2f = (b2 * scale2 + shift2)[:, :, None]

    hpk = _atom_encode(xT, embT)
    for l in range(LAYERS):
        agg2 = _get_sc_segsum()(hpk.reshape(-1),
                                edge_index).reshape(2, HID, NPAD)
        hpk = _mlp(hpk, agg2, W1f[l], b1f[l], W2f[l], b2f[l],
                   final=(l == LAYERS - 1))
    return _poolhead(hpk, batch3, Wh.T, bh[None, :], Wo.T, bo[None, :])


# speedup vs baseline: 8.2512x; 1.0370x over previous
import functools

import jax
import jax.numpy as jnp
from jax import lax
from jax.experimental import pallas as pl
from jax.experimental.pallas import tpu as pltpu
from jax.experimental.pallas import tpu_sc as plsc

N = 10000
E = 320000
NF = 9
VOCAB = 119
HID = 128
G = 256
LAYERS = 5

NPAD = 10240
NW = 32
NCORES = 2
FPW = 8
PKW = FPW // 2
EHALF = E // 2
ECH = 1280
NECH = EHALF // ECH
EUNROLL = 8

CB = 2048
NB = NPAD // CB


def _sc_segsum_body(hpk_hbm, edges_hbm, out_hbm, h_v, agg_v, idx0_v, idx1_v,
                    sem0, sem1, semh):
    wid = lax.axis_index("s") * NCORES + lax.axis_index("c")
    fg = wid // 2
    half = wid % 2
    ebase = half * EHALF

    def start(c, buf, sem):
        pltpu.async_copy(edges_hbm.at[:, pl.ds(ebase + c * ECH, ECH)], buf,
                         sem)

    def wait(buf, sem):
        pltpu.make_async_copy(edges_hbm.at[:, pl.ds(0, ECH)], buf, sem).wait()

    hsrc = hpk_hbm.at[pl.ds(fg * (PKW * NPAD), PKW * NPAD)]
    pltpu.async_copy(hsrc, h_v, semh)
    start(0, idx0_v, sem0)
    start(1, idx1_v, sem1)

    def zero_body(i, carry):
        for u in range(4):
            agg_v[pl.ds(i * 64 + u * 16, 16)] = jnp.zeros((16,), jnp.float32)
        return carry

    lax.fori_loop(0, FPW * NPAD // 64, zero_body, 0)
    pltpu.make_async_copy(hsrc, h_v, semh).wait()

    def process(idx_v):
        def edge_body(i, icarry):
            base = i * (16 * EUNROLL)
            srcs = [idx_v[0, pl.ds(base + u * 16, 16)] for u in range(EUNROLL)]
            dsts = [idx_v[1, pl.ds(base + u * 16, 16)] for u in range(EUNROLL)]
            words = [plsc.load_gather(h_v, [srcs[u] + (p * NPAD)])
                     for u in range(EUNROLL) for p in range(PKW)]
            for u in range(EUNROLL):
                for p in range(PKW):
                    wi = plsc.bitcast(words[u * PKW + p], jnp.int32)
                    lo = plsc.bitcast(wi << 16, jnp.float32)
                    hi = plsc.bitcast(wi & jnp.int32(-65536), jnp.float32)
                    plsc.addupdate_scatter(
                        agg_v, [dsts[u] + (2 * p * NPAD)], lo)
                    plsc.addupdate_scatter(
                        agg_v, [dsts[u] + ((2 * p + 1) * NPAD)], hi)
            return icarry

        lax.fori_loop(0, ECH // (16 * EUNROLL), edge_body, 0)

    def chunk_body(c2, carry):
        wait(idx0_v, sem0)
        process(idx0_v)

        @pl.when(2 * c2 + 2 < NECH)
        def _():
            start(2 * c2 + 2, idx0_v, sem0)

        wait(idx1_v, sem1)
        process(idx1_v)

        @pl.when(2 * c2 + 3 < NECH)
        def _():
            start(2 * c2 + 3, idx1_v, sem1)

        return carry

    lax.fori_loop(0, NECH // 2, chunk_body, 0)
    if NECH % 2:
        wait(idx0_v, sem0)
        process(idx0_v)
    out_off = half * (HID * NPAD) + fg * (FPW * NPAD)
    pltpu.sync_copy(agg_v, out_hbm.at[pl.ds(out_off, FPW * NPAD)])


@functools.lru_cache(maxsize=None)
def _get_sc_segsum():
    return pl.kernel(
        _sc_segsum_body,
        out_type=jax.ShapeDtypeStruct((2 * HID * NPAD,), jnp.float32),
        mesh=plsc.VectorSubcoreMesh(core_axis_name="c", subcore_axis_name="s"),
        compiler_params=pltpu.CompilerParams(needs_layout_passes=False),
        scratch_types=[
            pltpu.VMEM((PKW * NPAD,), jnp.float32),
            pltpu.VMEM((FPW * NPAD,), jnp.float32),
            pltpu.VMEM((2, ECH), jnp.int32),
            pltpu.VMEM((2, ECH), jnp.int32),
            pltpu.SemaphoreType.DMA,
            pltpu.SemaphoreType.DMA,
            pltpu.SemaphoreType.DMA,
        ],
    )


def _pack_pairs(z):
    zb = z.astype(jnp.bfloat16).reshape(HID // 2, 2, CB)
    u = lax.bitcast_convert_type(zb, jnp.uint16).astype(jnp.uint32)
    w = u[:, 0, :] | (u[:, 1, :] << jnp.uint32(16))
    return lax.bitcast_convert_type(w, jnp.float32)


def _unpack_pairs(wpk):
    wi = lax.bitcast_convert_type(wpk, jnp.uint32)
    lo = lax.bitcast_convert_type(wi << jnp.uint32(16), jnp.float32)
    hi = lax.bitcast_convert_type(wi & jnp.uint32(0xFFFF0000), jnp.float32)
    return jnp.stack([lo, hi], axis=1).reshape(HID, CB)


def _atom_body(xT_ref, embT_ref, pk_ref):
    iota = lax.broadcasted_iota(jnp.int32, (HID, CB), 0)
    acc = jnp.zeros((HID, CB), jnp.float32)
    for f in range(NF):
        onehot = (iota == xT_ref[f, :][None, :]).astype(jnp.float32)
        acc = acc + jnp.dot(embT_ref[f], onehot,
                            preferred_element_type=jnp.float32)
    pk_ref[...] = _pack_pairs(acc)


def _atom_encode(xT, embT):
    return pl.pallas_call(
        _atom_body,
        grid=(NB,),
        in_specs=[
            pl.BlockSpec((16, CB), lambda i: (0, i)),
            pl.BlockSpec((NF, HID, HID), lambda i: (0, 0, 0)),
        ],
        out_specs=pl.BlockSpec((HID // 2, CB), lambda i: (0, i)),
        out_shape=jax.ShapeDtypeStruct((HID // 2, NPAD), jnp.float32),
    )(xT, embT)


def _mlp_body(hpk_ref, a_ref, w1_ref, b1_ref, w2_ref, b2_ref, out_ref,
              *, final):
    z = _unpack_pairs(hpk_ref[...]) + a_ref[0] + a_ref[1]
    z = jnp.dot(w1_ref[...], z, preferred_element_type=jnp.float32) + b1_ref[...]
    z = jnp.maximum(z, 0.0)
    z = jnp.dot(w2_ref[...], z, preferred_element_type=jnp.float32) + b2_ref[...]
    if final:
        out_ref[...] = z
    else:
        out_ref[...] = _pack_pairs(jnp.maximum(z, 0.0))


def _mlp(hpk, agg2, w1, b1, w2, b2, final):
    orows = HID if final else HID // 2
    return pl.pallas_call(
        functools.partial(_mlp_body, final=final),
        grid=(NB,),
        in_specs=[
            pl.BlockSpec((HID // 2, CB), lambda i: (0, i)),
            pl.BlockSpec((2, HID, CB), lambda i: (0, 0, i)),
            pl.BlockSpec((HID, HID), lambda i: (0, 0)),
            pl.BlockSpec((HID, 1), lambda i: (0, 0)),
            pl.BlockSpec((HID, HID), lambda i: (0, 0)),
            pl.BlockSpec((HID, 1), lambda i: (0, 0)),
        ],
        out_specs=pl.BlockSpec((orows, CB), lambda i: (0, i)),
        out_shape=jax.ShapeDtypeStruct((orows, NPAD), jnp.float32),
    )(hpk, agg2, w1, b1, w2, b2)


def _poolhead_body(h_ref, b_ref, whT_ref, bh_ref, woT_ref, bo_ref, out_ref,
                   acc_ref):
    @pl.when(pl.program_id(0) == 0)
    def _():
        acc_ref[...] = jnp.zeros_like(acc_ref)

    iota = lax.broadcasted_iota(jnp.int32, (G, CB), 0)
    onehotT = (iota == b_ref[0, :, :]).astype(jnp.float32)
    acc_ref[...] += lax.dot_general(
        onehotT, h_ref[...], (((1,), (1,)), ((), ())),
        preferred_element_type=jnp.float32)

    @pl.when(pl.program_id(0) == NB - 1)
    def _():
        a = jnp.dot(acc_ref[...], whT_ref[...],
                    preferred_element_type=jnp.float32)
        a = jnp.maximum(a + bh_ref[...], 0.0)
        out_ref[...] = jnp.dot(a, woT_ref[...],
                               preferred_element_type=jnp.float32) + bo_ref[...]


def _poolhead(h, batch3, whT, bh2, woT, bo2):
    return pl.pallas_call(
        _poolhead_body,
        grid=(NB,),
        in_specs=[
            pl.BlockSpec((HID, CB), lambda i: (0, i)),
            pl.BlockSpec((1, 1, CB), lambda i: (i, 0, 0)),
            pl.BlockSpec((HID, HID), lambda i: (0, 0)),
            pl.BlockSpec((1, HID), lambda i: (0, 0)),
            pl.BlockSpec((HID, HID), lambda i: (0, 0)),
            pl.BlockSpec((1, HID), lambda i: (0, 0)),
        ],
        out_specs=pl.BlockSpec((G, HID), lambda i: (0, 0)),
        out_shape=jax.ShapeDtypeStruct((G, HID), jnp.float32),
        scratch_shapes=[pltpu.VMEM((G, HID), jnp.float32)],
    )(h, batch3, whT, bh2, woT, bo2)


def kernel(x, edge_index, batch, atom_emb, W1, b1, g1, be1, W2, b2,
           bn_g, bn_b, Wh, bh, Wo, bo):
    xT = jnp.pad(x, ((0, NPAD - N), (0, 0))).T
    xT = jnp.pad(xT, ((0, 16 - NF), (0, 0)))
    embT = jnp.pad(jnp.transpose(atom_emb, (0, 2, 1)),
                   ((0, 0), (0, 0), (0, HID - VOCAB)))
    batch3 = jnp.pad(batch, (0, NPAD - N),
                     constant_values=G + 1).reshape(NB, 1, CB)

    W1f = g1[:, :, None] * W1
    b1f = (b1 * g1 + be1)[:, :, None]
    scale2 = jnp.concatenate([bn_g, jnp.ones((1, HID), jnp.float32)], 0)
    shift2 = jnp.concatenate([bn_b, jnp.zeros((1, HID), jnp.float32)], 0)
    W2f = scale2[:, :, None] * W2
    b2f = (b2 * scale2 + shift2)[:, :, None]

    hpk = _atom_encode(xT, embT)
    for l in range(LAYERS):
        agg2 = _get_sc_segsum()(hpk.reshape(-1),
                                edge_index).reshape(2, HID, NPAD)
        hpk = _mlp(hpk, agg2, W1f[l], b1f[l], W2f[l], b2f[l],
                   final=(l == LAYERS - 1))
    return _poolhead(hpk, batch3, Wh.T, bh[None, :], Wo.T, bo[None, :])


# speedup vs baseline: 8.2612x; 1.0012x over previous
import functools

import jax
import jax.numpy as jnp
from jax import lax
from jax.experimental import pallas as pl
from jax.experimental.pallas import tpu as pltpu
from jax.experimental.pallas import tpu_sc as plsc

N = 10000
E = 320000
NF = 9
VOCAB = 119
HID = 128
G = 256
LAYERS = 5

NPAD = 10240
NW = 32
NCORES = 2
FPW = 8
PKW = FPW // 2
EHALF = E // 2
ECH = 1280
NECH = EHALF // ECH
EUNROLL = 8

CB = 2048
NB = NPAD // CB


def _sc_segsum_body(hpk_hbm, edges_hbm, out_hbm, h_v, agg_v, idx0_v, idx1_v,
                    idx2_v, sem0, sem1, sem2, semh):
    wid = lax.axis_index("s") * NCORES + lax.axis_index("c")
    fg = wid // 2
    half = wid % 2
    ebase = half * EHALF

    def start(c, buf, sem):
        pltpu.async_copy(edges_hbm.at[:, pl.ds(ebase + c * ECH, ECH)], buf,
                         sem)

    def wait(buf, sem):
        pltpu.make_async_copy(edges_hbm.at[:, pl.ds(0, ECH)], buf, sem).wait()

    hsrc = hpk_hbm.at[pl.ds(fg * (PKW * NPAD), PKW * NPAD)]
    pltpu.async_copy(hsrc, h_v, semh)
    start(0, idx0_v, sem0)
    start(1, idx1_v, sem1)

    def zero_body(i, carry):
        for u in range(4):
            agg_v[pl.ds(i * 64 + u * 16, 16)] = jnp.zeros((16,), jnp.float32)
        return carry

    lax.fori_loop(0, FPW * NPAD // 64, zero_body, 0)
    pltpu.make_async_copy(hsrc, h_v, semh).wait()

    def process(idx_v):
        def edge_body(i, icarry):
            base = i * (16 * EUNROLL)
            srcs = [idx_v[0, pl.ds(base + u * 16, 16)] for u in range(EUNROLL)]
            dsts = [idx_v[1, pl.ds(base + u * 16, 16)] for u in range(EUNROLL)]
            words = [plsc.load_gather(h_v, [srcs[u] + (p * NPAD)])
                     for u in range(EUNROLL) for p in range(PKW)]
            for u in range(EUNROLL):
                for p in range(PKW):
                    wi = plsc.bitcast(words[u * PKW + p], jnp.int32)
                    lo = plsc.bitcast(wi << 16, jnp.float32)
                    hi = plsc.bitcast(wi & jnp.int32(-65536), jnp.float32)
                    plsc.addupdate_scatter(
                        agg_v, [dsts[u] + (2 * p * NPAD)], lo)
                    plsc.addupdate_scatter(
                        agg_v, [dsts[u] + ((2 * p + 1) * NPAD)], hi)
            return icarry

        lax.fori_loop(0, ECH // (16 * EUNROLL), edge_body, 0)

    bufs = ((idx0_v, sem0), (idx1_v, sem1), (idx2_v, sem2))

    def chunk_body(c3, carry):
        for k, (buf, sem) in enumerate(bufs):
            wait(buf, sem)
            process(buf)

            @pl.when(3 * c3 + 3 + k < NECH)
            def _():
                start(3 * c3 + 3 + k, buf, sem)

        return carry

    start(2, idx2_v, sem2)
    lax.fori_loop(0, NECH // 3, chunk_body, 0)
    for k in range(NECH % 3):
        buf, sem = bufs[k]
        wait(buf, sem)
        process(buf)
    out_off = half * (HID * NPAD) + fg * (FPW * NPAD)
    pltpu.sync_copy(agg_v, out_hbm.at[pl.ds(out_off, FPW * NPAD)])


@functools.lru_cache(maxsize=None)
def _get_sc_segsum():
    return pl.kernel(
        _sc_segsum_body,
        out_type=jax.ShapeDtypeStruct((2 * HID * NPAD,), jnp.float32),
        mesh=plsc.VectorSubcoreMesh(core_axis_name="c", subcore_axis_name="s"),
        compiler_params=pltpu.CompilerParams(needs_layout_passes=False),
        scratch_types=[
            pltpu.VMEM((PKW * NPAD,), jnp.float32),
            pltpu.VMEM((FPW * NPAD,), jnp.float32),
            pltpu.VMEM((2, ECH), jnp.int32),
            pltpu.VMEM((2, ECH), jnp.int32),
            pltpu.VMEM((2, ECH), jnp.int32),
            pltpu.SemaphoreType.DMA,
            pltpu.SemaphoreType.DMA,
            pltpu.SemaphoreType.DMA,
            pltpu.SemaphoreType.DMA,
        ],
    )


def _pack_pairs(z):
    zb = z.astype(jnp.bfloat16).reshape(HID // 2, 2, CB)
    u = lax.bitcast_convert_type(zb, jnp.uint16).astype(jnp.uint32)
    w = u[:, 0, :] | (u[:, 1, :] << jnp.uint32(16))
    return lax.bitcast_convert_type(w, jnp.float32)


def _unpack_pairs(wpk):
    wi = lax.bitcast_convert_type(wpk, jnp.uint32)
    lo = lax.bitcast_convert_type(wi << jnp.uint32(16), jnp.float32)
    hi = lax.bitcast_convert_type(wi & jnp.uint32(0xFFFF0000), jnp.float32)
    return jnp.stack([lo, hi], axis=1).reshape(HID, CB)


def _atom_body(xT_ref, embT_ref, pk_ref):
    iota = lax.broadcasted_iota(jnp.int32, (HID, CB), 0)
    acc = jnp.zeros((HID, CB), jnp.float32)
    for f in range(NF):
        onehot = (iota == xT_ref[f, :][None, :]).astype(jnp.float32)
        acc = acc + jnp.dot(embT_ref[f], onehot,
                            preferred_element_type=jnp.float32)
    pk_ref[...] = _pack_pairs(acc)


def _atom_encode(xT, embT):
    return pl.pallas_call(
        _atom_body,
        grid=(NB,),
        in_specs=[
            pl.BlockSpec((16, CB), lambda i: (0, i)),
            pl.BlockSpec((NF, HID, HID), lambda i: (0, 0, 0)),
        ],
        out_specs=pl.BlockSpec((HID // 2, CB), lambda i: (0, i)),
        out_shape=jax.ShapeDtypeStruct((HID // 2, NPAD), jnp.float32),
    )(xT, embT)


def _mlp_body(hpk_ref, a_ref, w1_ref, b1_ref, w2_ref, b2_ref, out_ref,
              *, final):
    z = _unpack_pairs(hpk_ref[...]) + a_ref[0] + a_ref[1]
    z = jnp.dot(w1_ref[...], z, preferred_element_type=jnp.float32) + b1_ref[...]
    z = jnp.maximum(z, 0.0)
    z = jnp.dot(w2_ref[...], z, preferred_element_type=jnp.float32) + b2_ref[...]
    if final:
        out_ref[...] = z
    else:
        out_ref[...] = _pack_pairs(jnp.maximum(z, 0.0))


def _mlp(hpk, agg2, w1, b1, w2, b2, final):
    orows = HID if final else HID // 2
    return pl.pallas_call(
        functools.partial(_mlp_body, final=final),
        grid=(NB,),
        in_specs=[
            pl.BlockSpec((HID // 2, CB), lambda i: (0, i)),
            pl.BlockSpec((2, HID, CB), lambda i: (0, 0, i)),
            pl.BlockSpec((HID, HID), lambda i: (0, 0)),
            pl.BlockSpec((HID, 1), lambda i: (0, 0)),
            pl.BlockSpec((HID, HID), lambda i: (0, 0)),
            pl.BlockSpec((HID, 1), lambda i: (0, 0)),
        ],
        out_specs=pl.BlockSpec((orows, CB), lambda i: (0, i)),
        out_shape=jax.ShapeDtypeStruct((orows, NPAD), jnp.float32),
    )(hpk, agg2, w1, b1, w2, b2)


def _poolhead_body(h_ref, b_ref, whT_ref, bh_ref, woT_ref, bo_ref, out_ref,
                   acc_ref):
    @pl.when(pl.program_id(0) == 0)
    def _():
        acc_ref[...] = jnp.zeros_like(acc_ref)

    iota = lax.broadcasted_iota(jnp.int32, (G, CB), 0)
    onehotT = (iota == b_ref[0, :, :]).astype(jnp.float32)
    acc_ref[...] += lax.dot_general(
        onehotT, h_ref[...], (((1,), (1,)), ((), ())),
        preferred_element_type=jnp.float32)

    @pl.when(pl.program_id(0) == NB - 1)
    def _():
        a = jnp.dot(acc_ref[...], whT_ref[...],
                    preferred_element_type=jnp.float32)
        a = jnp.maximum(a + bh_ref[...], 0.0)
        out_ref[...] = jnp.dot(a, woT_ref[...],
                               preferred_element_type=jnp.float32) + bo_ref[...]


def _poolhead(h, batch3, whT, bh2, woT, bo2):
    return pl.pallas_call(
        _poolhead_body,
        grid=(NB,),
        in_specs=[
            pl.BlockSpec((HID, CB), lambda i: (0, i)),
            pl.BlockSpec((1, 1, CB), lambda i: (i, 0, 0)),
            pl.BlockSpec((HID, HID), lambda i: (0, 0)),
            pl.BlockSpec((1, HID), lambda i: (0, 0)),
            pl.BlockSpec((HID, HID), lambda i: (0, 0)),
            pl.BlockSpec((1, HID), lambda i: (0, 0)),
        ],
        out_specs=pl.BlockSpec((G, HID), lambda i: (0, 0)),
        out_shape=jax.ShapeDtypeStruct((G, HID), jnp.float32),
        scratch_shapes=[pltpu.VMEM((G, HID), jnp.float32)],
    )(h, batch3, whT, bh2, woT, bo2)


def kernel(x, edge_index, batch, atom_emb, W1, b1, g1, be1, W2, b2,
           bn_g, bn_b, Wh, bh, Wo, bo):
    xT = jnp.pad(x, ((0, NPAD - N), (0, 0))).T
    xT = jnp.pad(xT, ((0, 16 - NF), (0, 0)))
    embT = jnp.pad(jnp.transpose(atom_emb, (0, 2, 1)),
                   ((0, 0), (0, 0), (0, HID - VOCAB)))
    batch3 = jnp.pad(batch, (0, NPAD - N),
                     constant_values=G + 1).reshape(NB, 1, CB)

    W1f = g1[:, :, None] * W1
    b1f = (b1 * g1 + be1)[:, :, None]
    scale2 = jnp.concatenate([bn_g, jnp.ones((1, HID), jnp.float32)], 0)
    shift2 = jnp.concatenate([bn_b, jnp.zeros((1, HID), jnp.float32)], 0)
    W2f = scale2[:, :, None] * W2
    b2f = (b2 * scale2 + shift2)[:, :, None]

    hpk = _atom_encode(xT, embT)
    for l in range(LAYERS):
        agg2 = _get_sc_segsum()(hpk.reshape(-1),
                                edge_index).reshape(2, HID, NPAD)
        hpk = _mlp(hpk, agg2, W1f[l], b1f[l], W2f[l], b2f[l],
                   final=(l == LAYERS - 1))
    return _poolhead(hpk, batch3, Wh.T, bh[None, :], Wo.T, bo[None, :])
